# baseline (speedup 1.0000x reference)
import jax
import jax.numpy as jnp
from jax import lax
from jax.experimental import pallas as pl
from jax.experimental.pallas import tpu as pltpu
from jax.experimental.pallas import tpu_sc as plsc

NC = 2
NS = 16
NW = NC * NS
L = 16

_B, _S, _D = 1024, 200, 64
_FLAT = _B * _S
_PER_W = _FLAT // NW
_CH = _S
_NCH = _PER_W // _CH
_HALF = _CH // 2


def _make_kernel():
    mesh = plsc.VectorSubcoreMesh(
        core_axis_name="c", subcore_axis_name="s",
        num_cores=NC, num_subcores=NS)

    @pl.kernel(
        out_type=jax.ShapeDtypeStruct((_FLAT, _D), jnp.float32),
        mesh=mesh,
        compiler_params=pltpu.CompilerParams(use_tc_tiling_on_sc=False),
        scratch_types=[
            pltpu.VMEM((_NCH, 2, _HALF), jnp.int32),
            pltpu.VMEM((_S, _D), jnp.float32),
            pltpu.VMEM((_CH, _D), jnp.float32),
            pltpu.SemaphoreType.DMA,
        ],
    )
    def k(x_hbm, table_hbm, pe_hbm, out_hbm, idx_v, pe_v, rows_v, sem):
        wid = lax.axis_index("s") * NC + lax.axis_index("c")
        base = wid * _PER_W
        pltpu.sync_copy(x_hbm.at[wid], idx_v)
        pltpu.sync_copy(pe_hbm, pe_v)

        def chunk_body(kk, _):
            pltpu.async_copy(
                table_hbm.at[idx_v.at[kk, 0]],
                rows_v.at[pl.ds(0, _HALF)], sem).wait()
            pltpu.async_copy(
                table_hbm.at[idx_v.at[kk, 1]],
                rows_v.at[pl.ds(_HALF, _HALF)], sem).wait()

            def row_body(r, carry):
                for c in range(_D // L):
                    sl = pl.ds(c * L, L)
                    plsc.addupdate(rows_v.at[r, sl], pe_v[r, sl])
                return carry

            lax.fori_loop(0, _CH, row_body, 0)
            pltpu.sync_copy(rows_v, out_hbm.at[pl.ds(base + kk * _CH, _CH)])
            return _

        lax.fori_loop(0, _NCH, chunk_body, 0)

    return k


_kernel_call = _make_kernel()


def kernel(x, token_table, pe):
    idx = x.reshape(NW, _NCH, 2, _HALF)
    out = _kernel_call(idx, token_table, pe[:_S])
    return out.reshape(_B, _S, _D)


# speedup vs baseline: 1.0188x; 1.0188x over previous
import jax
import jax.numpy as jnp
from jax import lax
from jax.experimental import pallas as pl
from jax.experimental.pallas import tpu as pltpu
from jax.experimental.pallas import tpu_sc as plsc

NC = 2
NS = 16
NW = NC * NS
L = 16

_B, _S, _D = 1024, 200, 64
_ROWS_W = _B // NW


def _make_kernel():
    mesh = plsc.VectorSubcoreMesh(
        core_axis_name="c", subcore_axis_name="s",
        num_cores=NC, num_subcores=NS)

    @pl.kernel(
        out_type=jax.ShapeDtypeStruct((_B, _S, _D), jnp.float32),
        mesh=mesh,
        compiler_params=pltpu.CompilerParams(use_tc_tiling_on_sc=False),
        scratch_types=[
            pltpu.VMEM((_ROWS_W, _S), jnp.int32),
            pltpu.VMEM((_S, _D), jnp.float32),
            pltpu.VMEM((_S, _D), jnp.float32),
            pltpu.SemaphoreType.DMA,
        ],
    )
    def k(x_hbm, table_hbm, pe_hbm, out_hbm, idx_v, pe_v, rows_v, sem):
        wid = lax.axis_index("s") * NC + lax.axis_index("c")
        base = wid * _ROWS_W
        pltpu.sync_copy(x_hbm.at[pl.ds(base, _ROWS_W)], idx_v)
        pltpu.sync_copy(pe_hbm.at[pl.ds(0, _S)], pe_v)

        def chunk_body(kk, carry):
            pltpu.async_copy(table_hbm.at[idx_v.at[kk]], rows_v, sem).wait()

            def row_body(r, c2):
                for c in range(_D // L):
                    sl = pl.ds(c * L, L)
                    plsc.addupdate(rows_v.at[r, sl], pe_v[r, sl])
                return c2

            lax.fori_loop(0, _S, row_body, 0)
            pltpu.sync_copy(rows_v, out_hbm.at[base + kk])
            return carry

        lax.fori_loop(0, _ROWS_W, chunk_body, 0)

    return k


_kernel_call = _make_kernel()


def kernel(x, token_table, pe):
    return _kernel_call(x, token_table, pe)
